# baseline (speedup 1.0000x reference)
import functools
import math

import jax
import jax.numpy as jnp
from jax import lax
from jax.experimental import pallas as pl
from jax.experimental.pallas import tpu as pltpu
from jax.experimental.pallas import tpu_sc as plsc

VOCAB_SIZE = 1000000
D = 64
SCALE = math.sqrt(D)

NC = 2
NS = 16
NW = NC * NS

I, J = 4096, 200
IT, IL = I // 128, 128
JT, JS = J // 8, 8
G = IT * JT * JS
GPW = G // NW


def _embed_body(idx_hbm, table_hbm, out_hbm,
                idx_v, r0, r1, t0, t1,
                semg0, semg1, sems0, sems1):
    rbufs = (r0, r1)
    tbufs = (t0, t1)
    gsems = (semg0, semg1)
    ssems = (sems0, sems1)

    wid = lax.axis_index("s") * NC + lax.axis_index("c")
    base_g = wid * GPW

    pltpu.sync_copy(idx_hbm.at[pl.ds(base_g * 128, GPW * 128)], idx_v)

    def start_gather(gl, a):
        pltpu.async_copy(
            table_hbm.at[idx_v.at[pl.ds(gl * 128, 128)]], rbufs[a], gsems[a])

    def wait_gather(a):
        pltpu.make_async_copy(
            table_hbm.at[pl.ds(0, 128)], rbufs[a], gsems[a]).wait()

    def drain_stores(a):
        pltpu.make_async_copy(
            tbufs[a], out_hbm.at[pl.ds(0, 8), 0], ssems[a]).wait()

    iota = lax.iota(jnp.int32, 16)

    start_gather(0, 0)
    start_gather(1, 1)

    @pl.loop(0, GPW, step=2)
    def _(g0):
        for a in range(2):
            gl = g0 + a
            wait_gather(a)

            @pl.when(gl >= 2)
            def _():
                drain_stores(a)

            rb = rbufs[a]
            tb = tbufs[a]

            @pl.loop(0, D)
            def _(d):
                colv = jnp.full((16,), d, jnp.int32)
                dti = d >> 3
                dsi = d & 7
                for q in range(8):
                    rowv = iota + (q * 16)
                    val = plsc.load_gather(rb, [rowv, colv]) * SCALE
                    tb[dti, dsi, pl.ds(q * 16, 16)] = val

            g = base_g + gl
            jt = g >> 8
            it = (g >> 3) & 31
            js = g & 7
            row_base = jt * 64 + js * 8
            pltpu.async_copy(
                tb, out_hbm.at[pl.ds(row_base, 8), it], ssems[a])

            @pl.when(gl + 2 < GPW)
            def _():
                start_gather(gl + 2, a)

    drain_stores(0)
    drain_stores(1)


def kernel(x, table):
    assert x.shape == (I, J) and table.shape == (VOCAB_SIZE, D)
    xp = (x.astype(jnp.int32)
          .reshape(IT, IL, JT, JS)
          .transpose(2, 0, 3, 1)
          .reshape(-1))

    mesh = plsc.VectorSubcoreMesh(core_axis_name="c", subcore_axis_name="s")
    out4 = pl.kernel(
        _embed_body,
        out_type=jax.ShapeDtypeStruct((J * 8, IT, 8, IL), jnp.float32),
        mesh=mesh,
        compiler_params=pltpu.CompilerParams(
            use_tc_tiling_on_sc=False, needs_layout_passes=False),
        scratch_types=[
            pltpu.VMEM((GPW * 128,), jnp.int32),
            pltpu.VMEM((128, D), jnp.float32),
            pltpu.VMEM((128, D), jnp.float32),
            pltpu.VMEM((8, 8, 128), jnp.float32),
            pltpu.VMEM((8, 8, 128), jnp.float32),
            pltpu.SemaphoreType.DMA,
            pltpu.SemaphoreType.DMA,
            pltpu.SemaphoreType.DMA,
            pltpu.SemaphoreType.DMA,
        ],
    )(xp, table)
    return (out4.reshape(J, 8, IT, 8, IL)
            .transpose(2, 4, 0, 1, 3)
            .reshape(I, J, D))


# speedup vs baseline: 1.2870x; 1.2870x over previous
import functools
import math

import jax
import jax.numpy as jnp
from jax import lax
from jax.experimental import pallas as pl
from jax.experimental.pallas import tpu as pltpu
from jax.experimental.pallas import tpu_sc as plsc

VOCAB_SIZE = 1000000
D = 64
SCALE = math.sqrt(D)

NC = 2
NS = 16
NW = NC * NS

I, J = 4096, 200
IT, IL = I // 128, 128
JT, JS = J // 8, 8
G = IT * JT * JS
GPW = G // NW


def _embed_body(idx_hbm, table_hbm, out_hbm,
                idx_v, r0, r1, t0, t1,
                semg0, semg1, sems0, sems1):
    rbufs = (r0, r1)
    tbufs = (t0, t1)
    gsems = (semg0, semg1)
    ssems = (sems0, sems1)

    wid = lax.axis_index("s") * NC + lax.axis_index("c")
    base_g = wid * GPW

    pltpu.sync_copy(idx_hbm.at[pl.ds(base_g * 128, GPW * 128)], idx_v)

    def start_gather(gl, a):
        pltpu.async_copy(
            table_hbm.at[idx_v.at[pl.ds(gl * 128, 128)]], rbufs[a], gsems[a])

    def wait_gather(a):
        pltpu.make_async_copy(
            table_hbm.at[pl.ds(0, 128)], rbufs[a], gsems[a]).wait()

    def drain_stores(a):
        pltpu.make_async_copy(
            tbufs[a], out_hbm.at[pl.ds(0, 8), 0], ssems[a]).wait()

    iota = lax.iota(jnp.int32, 16)

    start_gather(0, 0)
    start_gather(1, 1)

    @pl.loop(0, GPW, step=2)
    def _(g0):
        for a in range(2):
            gl = g0 + a
            wait_gather(a)

            @pl.when(gl >= 2)
            def _():
                drain_stores(a)

            rb = rbufs[a]
            tb = tbufs[a]

            @pl.loop(0, D, step=4)
            def _(d0):
                vals = []
                for dd in range(4):
                    colv = jnp.full((16,), d0 + dd, jnp.int32)
                    for q in range(8):
                        rowv = iota + (q * 16)
                        vals.append(plsc.load_gather(rb, [rowv, colv]))
                for dd in range(4):
                    d = d0 + dd
                    dti = d >> 3
                    dsi = d & 7
                    for q in range(8):
                        tb[dti, dsi, pl.ds(q * 16, 16)] = (
                            vals[dd * 8 + q] * SCALE)

            g = base_g + gl
            jt = g >> 8
            it = (g >> 3) & 31
            js = g & 7
            row_base = jt * 64 + js * 8
            for dt in range(8):
                pltpu.async_copy(
                    tb.at[dt], out_hbm.at[row_base + dt, it], ssems[a])

            @pl.when(gl + 2 < GPW)
            def _():
                start_gather(gl + 2, a)

    drain_stores(0)
    drain_stores(1)


def kernel(x, table):
    assert x.shape == (I, J) and table.shape == (VOCAB_SIZE, D)
    xp = (x.astype(jnp.int32)
          .reshape(IT, IL, JT, JS)
          .transpose(2, 0, 3, 1)
          .reshape(-1))

    mesh = plsc.VectorSubcoreMesh(core_axis_name="c", subcore_axis_name="s")
    out4 = pl.kernel(
        _embed_body,
        out_type=jax.ShapeDtypeStruct((J * 8, IT, 8, IL), jnp.float32),
        mesh=mesh,
        compiler_params=pltpu.CompilerParams(
            use_tc_tiling_on_sc=False, needs_layout_passes=False),
        scratch_types=[
            pltpu.VMEM((GPW * 128,), jnp.int32),
            pltpu.VMEM((128, D), jnp.float32),
            pltpu.VMEM((128, D), jnp.float32),
            pltpu.VMEM((8, 8, 128), jnp.float32),
            pltpu.VMEM((8, 8, 128), jnp.float32),
            pltpu.SemaphoreType.DMA,
            pltpu.SemaphoreType.DMA,
            pltpu.SemaphoreType.DMA,
            pltpu.SemaphoreType.DMA,
        ],
    )(xp, table)
    return (out4.reshape(J, 8, IT, 8, IL)
            .transpose(2, 4, 0, 1, 3)
            .reshape(I, J, D))


# speedup vs baseline: 1.9175x; 1.4900x over previous
import functools
import math

import jax
import jax.numpy as jnp
from jax import lax
from jax.experimental import pallas as pl
from jax.experimental.pallas import tpu as pltpu
from jax.experimental.pallas import tpu_sc as plsc

VOCAB_SIZE = 1000000
D = 64
SCALE = math.sqrt(D)

NC = 2
NS = 16
NW = NC * NS

I, J = 4096, 200
IT, IL = I // 128, 128
JT, JS = J // 8, 8
G = IT * JT * JS
GPW = G // NW


def _embed_body(idx_hbm, table_hbm, out_hbm,
                idx_v, r0, r1, t0, t1, rm,
                semg0, semg1, sems0, sems1):
    rbufs = (r0, r1)
    tbufs = (t0, t1)
    gsems = (semg0, semg1)
    ssems = (sems0, sems1)

    wid = lax.axis_index("s") * NC + lax.axis_index("c")
    base_g = wid * GPW

    pltpu.sync_copy(idx_hbm.at[pl.ds(base_g * 128, GPW * 128)], idx_v)

    def start_gather(gl, a):
        pltpu.async_copy(
            table_hbm.at[idx_v.at[pl.ds(gl * 128, 128)]], rbufs[a], gsems[a])

    def wait_gather(a):
        pltpu.make_async_copy(
            table_hbm.at[pl.ds(0, 128)], rbufs[a], gsems[a]).wait()

    def drain_stores(a):
        for _ in range(8):
            pltpu.make_async_copy(
                tbufs[a].at[0], out_hbm.at[0, 0], ssems[a]).wait()

    iota = lax.iota(jnp.int32, 16)

    start_gather(0, 0)
    start_gather(1, 1)

    @pl.loop(0, GPW, step=2)
    def _(g0):
        for a in range(2):
            gl = g0 + a
            wait_gather(a)

            @pl.when(gl >= 2)
            def _():
                drain_stores(a)

            rb = rbufs[a]
            tb = tbufs[a]

            @pl.loop(0, 128, step=2)
            def _(il0):
                for s in range(2):
                    il = il0 + s
                    colv = jnp.full((16,), il, jnp.int32)
                    vals = [rb[il, pl.ds(q * 16, 16)] for q in range(4)]
                    for q in range(4):
                        plsc.store_scatter(
                            rm, [iota + q * 16, colv], vals[q] * SCALE)

            @pl.loop(0, D, step=2)
            def _(d0):
                for s in range(2):
                    d = d0 + s
                    dti = d >> 3
                    dsi = d & 7
                    vals = [rm[d, pl.ds(q2 * 16, 16)] for q2 in range(8)]
                    for q2 in range(8):
                        tb[dti, dsi, pl.ds(q2 * 16, 16)] = vals[q2]

            g = base_g + gl
            jt = g >> 8
            it = (g >> 3) & 31
            js = g & 7
            row_base = jt * 64 + js * 8
            for dt in range(8):
                pltpu.async_copy(
                    tb.at[dt], out_hbm.at[row_base + dt, it], ssems[a])

            @pl.when(gl + 2 < GPW)
            def _():
                start_gather(gl + 2, a)

    drain_stores(0)
    drain_stores(1)


def kernel(x, table):
    assert x.shape == (I, J) and table.shape == (VOCAB_SIZE, D)
    xp = (x.astype(jnp.int32)
          .reshape(IT, IL, JT, JS)
          .transpose(2, 0, 3, 1)
          .reshape(-1))

    mesh = plsc.VectorSubcoreMesh(core_axis_name="c", subcore_axis_name="s")
    out4 = pl.kernel(
        _embed_body,
        out_type=jax.ShapeDtypeStruct((J * 8, IT, 8, IL), jnp.float32),
        mesh=mesh,
        compiler_params=pltpu.CompilerParams(
            use_tc_tiling_on_sc=False, needs_layout_passes=False),
        scratch_types=[
            pltpu.VMEM((GPW * 128,), jnp.int32),
            pltpu.VMEM((128, D), jnp.float32),
            pltpu.VMEM((128, D), jnp.float32),
            pltpu.VMEM((8, 8, 128), jnp.float32),
            pltpu.VMEM((8, 8, 128), jnp.float32),
            pltpu.VMEM((D, 136), jnp.float32),
            pltpu.SemaphoreType.DMA,
            pltpu.SemaphoreType.DMA,
            pltpu.SemaphoreType.DMA,
            pltpu.SemaphoreType.DMA,
        ],
    )(xp, table)
    return (out4.reshape(J, 8, IT, 8, IL)
            .transpose(2, 4, 0, 1, 3)
            .reshape(I, J, D))
